# baseline (speedup 1.0000x reference)
import jax
import jax.numpy as jnp

K = 20


def _lrelu(x):
    return jnp.where(x >= 0, x, 0.2 * x)


def _bn(x, g, b, ca=1):
    axes = tuple(i for i in range(x.ndim) if i != ca)
    m = jnp.mean(x, axis=axes, keepdims=True)
    v = jnp.var(x, axis=axes, keepdims=True)
    sh = [1] * x.ndim
    sh[ca] = -1
    return (x - m) / jnp.sqrt(v + 1e-5) * g.reshape(sh) + b.reshape(sh)


def _knn_graph(x, k):
    xt = jnp.swapaxes(x, 1, 2)
    inner = -2.0 * jnp.einsum('bnc,bcm->bnm', xt, x)
    xx = jnp.sum(x ** 2, axis=1, keepdims=True)
    pd = -xx - inner - jnp.swapaxes(xx, 1, 2)
    idx = jax.lax.top_k(pd, k)[1]
    B = x.shape[0]
    feat = xt[jnp.arange(B)[:, None, None], idx]
    xr = xt[:, :, None, :]
    out = jnp.concatenate([feat - xr, jnp.broadcast_to(xr, feat.shape)], axis=-1)
    return jnp.transpose(out, (0, 3, 1, 2))


def _edge_block(x, W, g, b):
    f = _knn_graph(x, K)
    h = jnp.einsum('oc,bcnk->bonk', W, f)
    return jnp.max(_lrelu(_bn(h, g, b, 1)), axis=-1)


def kernel(points, W1, g1, b1, W2, g2, b2, W3, g3, b3, W4, g4, b4, W5, g5, b5, Wf1, gf1, bf1, Wf2, bfc2, gf2, bf2, Wf3, bfc3):
    o1 = _edge_block(points, W1, g1, b1)
    o2 = _edge_block(o1, W2, g2, b2)
    o3 = _edge_block(o2, W3, g3, b3)
    o4 = _edge_block(o3, W4, g4, b4)
    cat = jnp.concatenate([o1, o2, o3, o4], axis=1)
    h = jnp.einsum('oc,bcn->bon', W5, cat)
    h = jnp.max(_lrelu(_bn(h, g5, b5, 1)), axis=-1)
    h = h @ Wf1.T
    h = _lrelu(_bn(h, gf1, bf1, 1))
    h = h @ Wf2.T + bfc2
    h = _lrelu(_bn(h, gf2, bf2, 1))
    return h @ Wf3.T + bfc3


# speedup vs baseline: 11.9716x; 11.9716x over previous
import functools

import jax
import jax.numpy as jnp
from jax import lax
from jax.experimental import pallas as pl
from jax.experimental.pallas import tpu as pltpu
from jax.experimental.pallas import tpu_sc as plsc

K = 20
N = 1024
B = 16
D = B * N
NW = 32
DPW = D // NW
CH = 512
NEG = -3.0e38
EPS = 1e-5


def _knn_body(xt_ref, x_ref, idx_ref):
    b = pl.program_id(0)
    xt = xt_ref[0]
    x = x_ref[0]
    gram = lax.dot_general(xt, x, (((1,), (0,)), ((), ())),
                           preferred_element_type=jnp.float32)
    sqr = jnp.sum(xt * xt, axis=1, keepdims=True)
    sqc = jnp.sum(x * x, axis=0, keepdims=True)
    pdm = 2.0 * gram - sqr - sqc
    li = lax.broadcasted_iota(jnp.int32, (N, N), 1)
    off = b * N
    for t in range(K):
        rmax = jnp.max(pdm, axis=1, keepdims=True)
        cand = jnp.where(pdm == rmax, li, jnp.int32(N))
        amax = jnp.min(cand, axis=1, keepdims=True)
        pdm = jnp.where(li == amax, NEG, pdm)
        idx_ref[0, :, pl.ds(t, 1)] = amax + off
    del x


def _knn(xt, x):
    Cp = xt.shape[2]
    return pl.pallas_call(
        _knn_body,
        grid=(B,),
        in_specs=[
            pl.BlockSpec((1, N, Cp), lambda b: (b, 0, 0)),
            pl.BlockSpec((1, Cp, N), lambda b: (b, 0, 0)),
        ],
        out_specs=pl.BlockSpec((1, N, K), lambda b: (b, 0, 0)),
        out_shape=jax.ShapeDtypeStruct((B, N, K), jnp.int32),
    )(xt, x)


def _sc_gather(xtab, idxk, G):
    Cp = xtab.shape[1]
    ng = DPW // G
    mesh = plsc.VectorSubcoreMesh(core_axis_name="c", subcore_axis_name="s")

    @functools.partial(
        pl.kernel,
        mesh=mesh,
        out_type=jax.ShapeDtypeStruct((K, D, Cp), jnp.float32),
        scratch_types=[
            pltpu.VMEM((DPW,), jnp.int32),
            pltpu.VMEM((G, Cp), jnp.float32),
            pltpu.SemaphoreType.DMA,
        ],
    )
    def k(xtab_hbm, idx_hbm, ft_hbm, idx_v, rows_v, sem):
        wid = lax.axis_index("s") * 2 + lax.axis_index("c")
        base_w = wid * DPW

        @pl.loop(0, K)
        def _k(kk):
            pltpu.sync_copy(idx_hbm.at[kk, pl.ds(base_w, DPW)], idx_v)

            @pl.loop(0, ng)
            def _g(j):
                pltpu.async_copy(
                    xtab_hbm.at[idx_v.at[pl.ds(j * G, G)]], rows_v, sem
                ).wait()
                pltpu.sync_copy(rows_v,
                                ft_hbm.at[kk, pl.ds(base_w + j * G, G)])

    return k(xtab, idxk)


def _edge_h_make(C, Op):
    def body(ft_ref, xt_ref, wt_ref, hmx_ref, s_ref):
        step = pl.program_id(0) * 2 + pl.program_id(1)
        xtc = xt_ref[...][:, :C]
        hm = None
        for kk in range(K):
            feat = ft_ref[kk][:, :C]
            f = jnp.concatenate([feat - xtc, xtc], axis=1)
            h = jnp.dot(f, wt_ref[...],
                        preferred_element_type=jnp.float32)
            if hm is None:
                hm, hs = h, h
            else:
                hm = jnp.maximum(hm, h)
                hs = hs + h
        hmx_ref[...] = hm
        rows = jnp.concatenate(
            [jnp.sum(hs, axis=0, keepdims=True),
             jnp.zeros((7, Op), jnp.float32)], axis=0)

        @pl.when(step == 0)
        def _():
            s_ref[...] = jnp.zeros_like(s_ref)

        s_ref[...] += rows
    return body


def _edge_h(ft, xtab, wt, C):
    Cp = xtab.shape[1]
    Op = wt.shape[1]
    return pl.pallas_call(
        _edge_h_make(C, Op),
        grid=(B, 2),
        in_specs=[
            pl.BlockSpec((K, CH, Cp), lambda b, c: (0, b * 2 + c, 0)),
            pl.BlockSpec((CH, Cp), lambda b, c: (b * 2 + c, 0)),
            pl.BlockSpec((2 * C, Op), lambda b, c: (0, 0)),
        ],
        out_specs=[
            pl.BlockSpec((CH, Op), lambda b, c: (b * 2 + c, 0)),
            pl.BlockSpec((8, Op), lambda b, c: (0, 0)),
        ],
        out_shape=[
            jax.ShapeDtypeStruct((D, Op), jnp.float32),
            jax.ShapeDtypeStruct((8, Op), jnp.float32),
        ],
    )(ft, xtab, wt)


def _edge_v_make(C, Op):
    def body(ft_ref, xt_ref, wt_ref, s1_ref, s2_ref):
        step = pl.program_id(0) * 2 + pl.program_id(1)
        m = s1_ref[0:1, :] / jnp.float32(D * K)
        xtc = xt_ref[...][:, :C]
        hq = None
        for kk in range(K):
            feat = ft_ref[kk][:, :C]
            f = jnp.concatenate([feat - xtc, xtc], axis=1)
            h = jnp.dot(f, wt_ref[...], preferred_element_type=jnp.float32)
            dlt = h - m
            hq = dlt * dlt if hq is None else hq + dlt * dlt
        rows = jnp.concatenate(
            [jnp.sum(hq, axis=0, keepdims=True),
             jnp.zeros((7, Op), jnp.float32)], axis=0)

        @pl.when(step == 0)
        def _():
            s2_ref[...] = jnp.zeros_like(s2_ref)

        s2_ref[...] += rows
    return body


def _edge_v(ft, xtab, wt, s1, C):
    Cp = xtab.shape[1]
    Op = wt.shape[1]
    return pl.pallas_call(
        _edge_v_make(C, Op),
        grid=(B, 2),
        in_specs=[
            pl.BlockSpec((K, CH, Cp), lambda b, c: (0, b * 2 + c, 0)),
            pl.BlockSpec((CH, Cp), lambda b, c: (b * 2 + c, 0)),
            pl.BlockSpec((2 * C, Op), lambda b, c: (0, 0)),
            pl.BlockSpec((8, Op), lambda b, c: (0, 0)),
        ],
        out_specs=pl.BlockSpec((8, Op), lambda b, c: (0, 0)),
        out_shape=jax.ShapeDtypeStruct((8, Op), jnp.float32),
    )(ft, xtab, wt, s1)


def _norm_body(mx_ref, s_ref, s2_ref, g_ref, bb_ref, out_ref):
    s = s_ref[...]
    cnt = jnp.float32(D * K)
    m = s[0:1, :] / cnt
    v = s2_ref[0:1, :] / cnt
    den = jnp.sqrt(v + EPS)
    h = (mx_ref[...] - m) / den * g_ref[...] + bb_ref[...]
    out_ref[...] = jnp.where(h >= 0, h, 0.2 * h)


def _block_norm(mx, s, s2, g, bb):
    Op = mx.shape[1]
    return pl.pallas_call(
        _norm_body,
        grid=(D // CH,),
        in_specs=[
            pl.BlockSpec((CH, Op), lambda c: (c, 0)),
            pl.BlockSpec((8, Op), lambda c: (0, 0)),
            pl.BlockSpec((8, Op), lambda c: (0, 0)),
            pl.BlockSpec((1, Op), lambda c: (0, 0)),
            pl.BlockSpec((1, Op), lambda c: (0, 0)),
        ],
        out_specs=pl.BlockSpec((CH, Op), lambda c: (c, 0)),
        out_shape=jax.ShapeDtypeStruct((D, Op), jnp.float32),
    )(mx, s, s2, g.reshape(1, Op), bb.reshape(1, Op))


def _edge_block(xtab, W, g, bb, G):
    Cp = xtab.shape[1]
    C = W.shape[1] // 2
    O = W.shape[0]
    Op = max(128, O)
    wt = jnp.transpose(W)
    if O < Op:
        wt = jnp.pad(wt, ((0, 0), (0, Op - O)))
        g = jnp.pad(g, (0, Op - O), constant_values=1.0)
        bb = jnp.pad(bb, (0, Op - O))
    xt3 = xtab.reshape(B, N, Cp)
    idx = _knn(xt3, jnp.swapaxes(xt3, 1, 2))
    idxk = jnp.transpose(idx.reshape(D, K))
    ft = _sc_gather(xtab, idxk, G)
    mx, s = _edge_h(ft, xtab, wt, C)
    s2 = _edge_v(ft, xtab, wt, s, C)
    return _block_norm(mx, s, s2, g, bb)


def _conv5_body(o1_ref, o2_ref, o3_ref, o4_ref, w1_ref, w2_ref, w3_ref, w4_ref,
                gmax_ref, s_ref):
    b = pl.program_id(0)
    h = jnp.dot(o1_ref[0], w1_ref[...], preferred_element_type=jnp.float32)
    h += jnp.dot(o2_ref[0], w2_ref[...], preferred_element_type=jnp.float32)
    h += jnp.dot(o3_ref[0], w3_ref[...], preferred_element_type=jnp.float32)
    h += jnp.dot(o4_ref[0], w4_ref[...], preferred_element_type=jnp.float32)
    gmax_ref[0] = jnp.max(h, axis=0, keepdims=True)
    rows = jnp.concatenate(
        [jnp.sum(h, axis=0, keepdims=True),
         jnp.sum(h * h, axis=0, keepdims=True)], axis=0)
    rows = jnp.concatenate([rows, jnp.zeros((6, rows.shape[1]), jnp.float32)],
                           axis=0)

    @pl.when(b == 0)
    def _():
        s_ref[...] = jnp.zeros_like(s_ref)

    s_ref[...] += rows


def _conv5(o1, o2, o3, o4, w5_t):
    w1, w2, w3, w4 = (w5_t[0:64], w5_t[64:128], w5_t[128:256], w5_t[256:512])
    return pl.pallas_call(
        _conv5_body,
        grid=(B,),
        in_specs=[
            pl.BlockSpec((1, N, 64), lambda b: (b, 0, 0)),
            pl.BlockSpec((1, N, 64), lambda b: (b, 0, 0)),
            pl.BlockSpec((1, N, 128), lambda b: (b, 0, 0)),
            pl.BlockSpec((1, N, 256), lambda b: (b, 0, 0)),
            pl.BlockSpec((64, 1024), lambda b: (0, 0)),
            pl.BlockSpec((64, 1024), lambda b: (0, 0)),
            pl.BlockSpec((128, 1024), lambda b: (0, 0)),
            pl.BlockSpec((256, 1024), lambda b: (0, 0)),
        ],
        out_specs=[
            pl.BlockSpec((1, 1, 1024), lambda b: (b, 0, 0)),
            pl.BlockSpec((8, 1024), lambda b: (0, 0)),
        ],
        out_shape=[
            jax.ShapeDtypeStruct((B, 1, 1024), jnp.float32),
            jax.ShapeDtypeStruct((8, 1024), jnp.float32),
        ],
    )(o1, o2, o3, o4, w1, w2, w3, w4)


def _head_body(gm_ref, s_ref, g5_ref, b5_ref, w1_ref, g1_ref, b1_ref,
               w2_ref, c2_ref, g2_ref, b2_ref, w3_ref, c3_ref, out_ref):
    s = s_ref[...]
    cnt = jnp.float32(B * N)
    m = s[0:1, :] / cnt
    v = s[1:2, :] / cnt - m * m
    h = (gm_ref[...] - m) / jnp.sqrt(v + EPS) * g5_ref[...] + b5_ref[...]
    h = jnp.where(h >= 0, h, 0.2 * h)

    def fc_bn(h, w_ref, g_ref, b_ref, c=None):
        y = jnp.dot(h, w_ref[...], preferred_element_type=jnp.float32)
        if c is not None:
            y = y + c[...]
        mm = jnp.sum(y, axis=0, keepdims=True) / B
        d = y - mm
        vv = jnp.sum(d * d, axis=0, keepdims=True) / B
        y = (y - mm) / jnp.sqrt(vv + EPS) * g_ref[...] + b_ref[...]
        return jnp.where(y >= 0, y, 0.2 * y)

    h = fc_bn(h, w1_ref, g1_ref, b1_ref)
    h = fc_bn(h, w2_ref, g2_ref, b2_ref, c2_ref)
    out_ref[...] = jnp.dot(h, w3_ref[...],
                           preferred_element_type=jnp.float32) + c3_ref[...]


def _head(gm, s, g5, b5, wf1_t, gf1, bf1, wf2_t, bfc2, gf2, bf2, wf3_t, bfc3):
    full = lambda shape: pl.BlockSpec(shape, lambda: tuple(0 for _ in shape))
    args = [gm, s, g5.reshape(1, -1), b5.reshape(1, -1),
            wf1_t, gf1.reshape(1, -1), bf1.reshape(1, -1),
            wf2_t, bfc2.reshape(1, -1), gf2.reshape(1, -1), bf2.reshape(1, -1),
            wf3_t, bfc3.reshape(1, -1)]
    return pl.pallas_call(
        _head_body,
        in_specs=[full(a.shape) for a in args],
        out_specs=full((B, 40)),
        out_shape=jax.ShapeDtypeStruct((B, 40), jnp.float32),
    )(*args)


def kernel(points, W1, g1, b1, W2, g2, b2, W3, g3, b3, W4, g4, b4,
           W5, g5, b5, Wf1, gf1, bf1, Wf2, bfc2, gf2, bf2, Wf3, bfc3):
    xt = jnp.swapaxes(points, 1, 2).reshape(D, 3)
    x0 = jnp.pad(xt, ((0, 0), (0, 125)))
    x1 = _edge_block(x0, W1, g1, b1, G=256)
    x2 = _edge_block(x1, W2, g2, b2, G=256)
    x3 = _edge_block(x2, W3, g3, b3, G=256)
    x4 = _edge_block(x3, W4, g4, b4, G=256)
    o1 = x1[:, :64].reshape(B, N, 64)
    o2 = x2[:, :64].reshape(B, N, 64)
    o3 = x3.reshape(B, N, 128)
    o4 = x4.reshape(B, N, 256)
    gm, s = _conv5(o1, o2, o3, o4, jnp.transpose(W5))
    gm = gm.reshape(B, 1024)
    return _head(gm, s, g5, b5, jnp.transpose(Wf1), gf1, bf1,
                 jnp.transpose(Wf2), bfc2, gf2, bf2, jnp.transpose(Wf3), bfc3)
